# Initial kernel scaffold; baseline (speedup 1.0000x reference)
import jax
import jax.numpy as jnp
from jax.experimental import pallas as pl


def kernel(features, edge_index, rel_types, W1, W1_self, b1, W2, W2_self, b2):
    raise NotImplementedError("write your pallas kernel here")


# speedup vs baseline: 15.5147x; 15.5147x over previous
import functools

import jax
import jax.numpy as jnp
from jax import lax
from jax.experimental import pallas as pl
from jax.experimental.pallas import tpu as pltpu
from jax.experimental.pallas import tpu_sc as plsc

N_NODES = 10000
N_EDGES = 320000
DIM = 128

NUM_CORES = 2
NUM_SUBCORES = 16
NW = NUM_CORES * NUM_SUBCORES
CHUNK = 128
CH_PER_W = 79
E_PAD = NW * CH_PER_W * CHUNK
ACC_ROWS = 10240
ZROWS = ACC_ROWS // NUM_SUBCORES

BN = 2000
NB = N_NODES // BN


def _transform(x, wall):
    def body(x_ref, w_ref, o_ref):
        o_ref[0] = jnp.dot(x_ref[...], w_ref[0],
                           preferred_element_type=jnp.float32)

    return pl.pallas_call(
        body,
        grid=(NB, 9),
        in_specs=[
            pl.BlockSpec((BN, DIM), lambda i, r: (i, 0)),
            pl.BlockSpec((1, DIM, DIM), lambda i, r: (r, 0, 0)),
        ],
        out_specs=pl.BlockSpec((1, BN, DIM), lambda i, r: (r, i, 0)),
        out_shape=jax.ShapeDtypeStruct((9, N_NODES, DIM), jnp.float32),
    )(x, wall)


def _mid(parts, t_prev, b, wall):
    def body(p_ref, t_ref, b_ref, w_ref, o_ref):
        h = p_ref[0] + p_ref[1] + t_ref[0] + b_ref[...]
        h = jnp.maximum(h, 0.0)
        o_ref[0] = jnp.dot(h, w_ref[0], preferred_element_type=jnp.float32)

    return pl.pallas_call(
        body,
        grid=(NB, 9),
        in_specs=[
            pl.BlockSpec((2, BN, DIM), lambda i, r: (0, i, 0)),
            pl.BlockSpec((1, BN, DIM), lambda i, r: (8, i, 0)),
            pl.BlockSpec((1, DIM), lambda i, r: (0, 0)),
            pl.BlockSpec((1, DIM, DIM), lambda i, r: (r, 0, 0)),
        ],
        out_specs=pl.BlockSpec((1, BN, DIM), lambda i, r: (r, i, 0)),
        out_shape=jax.ShapeDtypeStruct((9, N_NODES, DIM), jnp.float32),
    )(parts, t_prev, b, wall)


def _final(parts, t_prev, b):
    def body(p_ref, t_ref, b_ref, o_ref):
        o_ref[...] = p_ref[0] + p_ref[1] + t_ref[0] + b_ref[...]

    return pl.pallas_call(
        body,
        grid=(NB,),
        in_specs=[
            pl.BlockSpec((2, BN, DIM), lambda i: (0, i, 0)),
            pl.BlockSpec((1, BN, DIM), lambda i: (8, i, 0)),
            pl.BlockSpec((1, DIM), lambda i: (0, 0)),
        ],
        out_specs=pl.BlockSpec((BN, DIM), lambda i: (i, 0)),
        out_shape=jax.ShapeDtypeStruct((N_NODES, DIM), jnp.float32),
    )(parts, t_prev, b)


def _sc_gather_scatter(t_flat, gidx, dst, zblock):
    mesh = plsc.VectorSubcoreMesh(core_axis_name="c", subcore_axis_name="s")

    @functools.partial(
        pl.kernel,
        mesh=mesh,
        out_type=jax.ShapeDtypeStruct((NUM_CORES, ACC_ROWS, DIM), jnp.float32),
        scratch_types=[
            pltpu.VMEM((CH_PER_W, CHUNK), jnp.int32),
            pltpu.VMEM((CH_PER_W, CHUNK), jnp.int32),
            pltpu.VMEM((CHUNK, DIM), jnp.float32),
            pltpu.VMEM_SHARED((ACC_ROWS, DIM), jnp.float32),
            pltpu.SemaphoreType.DMA,
        ],
    )
    def k(t_hbm, gidx_hbm, dst_hbm, z_hbm, out_hbm,
          gidx_v, dst_v, rows_v, acc_sh, sem):
        cid = lax.axis_index("c")
        sid = lax.axis_index("s")
        wid = cid * NUM_SUBCORES + sid

        pltpu.sync_copy(gidx_hbm.at[wid], gidx_v)
        pltpu.sync_copy(dst_hbm.at[wid], dst_v)

        pltpu.sync_copy(z_hbm, rows_v)
        @pl.loop(0, ZROWS // CHUNK)
        def _(z):
            pltpu.sync_copy(
                rows_v, acc_sh.at[pl.ds(sid * ZROWS + z * CHUNK, CHUNK)])
        plsc.subcore_barrier()

        @pl.loop(0, CH_PER_W)
        def _(j):
            pltpu.async_copy(t_hbm.at[gidx_v.at[j]], rows_v, sem).wait()
            pltpu.sync_copy(rows_v, acc_sh.at[dst_v.at[j]], add=True)
        plsc.subcore_barrier()

        pltpu.sync_copy(acc_sh.at[pl.ds(sid * ZROWS, ZROWS)],
                        out_hbm.at[cid, pl.ds(sid * ZROWS, ZROWS)])

    return k(t_flat, gidx, dst, zblock)


def kernel(features, edge_index, rel_types, W1, W1_self, b1, W2, W2_self, b2):
    x = features.astype(jnp.float32)
    src = edge_index[0].astype(jnp.int32)
    dst = edge_index[1].astype(jnp.int32)
    rel = rel_types.astype(jnp.int32)

    gidx = rel * N_NODES + src
    pad = E_PAD - N_EDGES
    gidx = jnp.concatenate([gidx, jnp.zeros((pad,), jnp.int32)])
    gidx = gidx.reshape(NW, CH_PER_W, CHUNK)
    dsti = jnp.concatenate([dst, jnp.full((pad,), N_NODES, jnp.int32)])
    dsti = dsti.reshape(NW, CH_PER_W, CHUNK)
    zblock = jnp.zeros((CHUNK, DIM), jnp.float32)

    wall1 = jnp.concatenate([W1, W1_self[None]], axis=0)
    wall2 = jnp.concatenate([W2, W2_self[None]], axis=0)

    t1 = _transform(x, wall1)
    p1 = _sc_gather_scatter(t1.reshape(9 * N_NODES, DIM), gidx, dsti, zblock)
    t2 = _mid(p1, t1, b1.reshape(1, DIM), wall2)
    p2 = _sc_gather_scatter(t2.reshape(9 * N_NODES, DIM), gidx, dsti, zblock)
    return _final(p2, t2, b2.reshape(1, DIM))
